# baseline (speedup 1.0000x reference)
import functools
import math

import jax
import jax.numpy as jnp
from jax import lax
from jax.experimental import pallas as pl
from jax.experimental.pallas import tpu as pltpu
from jax.experimental.pallas import tpu_sc as plsc

D_MODEL = 128
SCALE = math.sqrt(D_MODEL)

_NUM_CORES = 2
_NUM_SUBCORES = 16
_NW = _NUM_CORES * _NUM_SUBCORES

_NBUF = 3
_LA = 2
_SPLIT = 128


@functools.partial(jax.jit, static_argnums=(2, 3))
def _sc_gather(x, table, xrows, seq):
    d = table.shape[1]
    n_rows = xrows * seq
    rows_per_w = xrows // _NW
    n_chunks = rows_per_w
    b_per_w = rows_per_w * seq
    lag = _NBUF - _LA
    steady = ((n_chunks - lag - _LA) // _NBUF) * _NBUF
    tail_start = lag + steady
    assert steady > 0 and n_chunks >= _NBUF + lag

    mesh = plsc.VectorSubcoreMesh(
        core_axis_name="c", subcore_axis_name="s",
        num_cores=_NUM_CORES, num_subcores=_NUM_SUBCORES,
    )

    @functools.partial(
        pl.kernel,
        out_type=jax.ShapeDtypeStruct((n_rows, d), jnp.float32),
        mesh=mesh,
        scratch_types=[
            pltpu.VMEM((rows_per_w, seq), jnp.int32),
            pltpu.VMEM((_NBUF, seq, d), jnp.float32),
        ] + [pltpu.SemaphoreType.DMA] * (2 * _NBUF),
    )
    def gather_kernel(idx_hbm, tab_hbm, out_hbm, idx_v, rows_v, *sems):
        sem_g = sems[:_NBUF]
        sem_o = sems[_NBUF:]
        wid = lax.axis_index("s") * _NUM_CORES + lax.axis_index("c")
        base = wid * b_per_w

        pltpu.sync_copy(idx_hbm.at[pl.ds(wid * rows_per_w, rows_per_w)],
                        idx_v)

        def gather_parts(i, b):
            yield (tab_hbm.at[idx_v.at[i, pl.ds(0, _SPLIT)]],
                   rows_v.at[b, pl.ds(0, _SPLIT)], sem_g[b])
            yield (tab_hbm.at[idx_v.at[i, pl.ds(_SPLIT, seq - _SPLIT)]],
                   rows_v.at[b, pl.ds(_SPLIT, seq - _SPLIT)], sem_g[b])

        def start_gather(i, b):
            for src, dst, sem in gather_parts(i, b):
                pltpu.async_copy(src, dst, sem)

        def wait_gather(i, b):
            for src, dst, sem in gather_parts(i, b):
                pltpu.make_async_copy(src, dst, sem).wait()

        def start_out(i, b):
            pltpu.async_copy(
                rows_v.at[b], out_hbm.at[pl.ds(base + i * seq, seq)],
                sem_o[b])

        def wait_out(i, b):
            pltpu.make_async_copy(
                rows_v.at[b], out_hbm.at[pl.ds(base + i * seq, seq)],
                sem_o[b]).wait()

        def scale_buf(b):
            @plsc.parallel_loop(0, seq, unroll=4)
            def _(r):
                for c in range(d // 16):
                    sl = pl.ds(c * 16, 16)
                    rows_v[b, r, sl] = rows_v[b, r, sl] * SCALE

        def step(i, slot, head=False, tail=False):
            gslot = (slot + _LA) % _NBUF
            if not head:
                wait_out(i - lag, gslot)
            if not tail:
                start_gather(i + _LA, gslot)
            wait_gather(i, slot)
            scale_buf(slot)
            start_out(i, slot)

        for j in range(_LA):
            start_gather(j, j % _NBUF)
        for i in range(lag):
            step(i, i % _NBUF, head=True)

        def steady_body(k, carry):
            i0 = lag + k * _NBUF
            for b in range(_NBUF):
                step(i0 + b, (lag + b) % _NBUF)
            return carry

        lax.fori_loop(0, steady // _NBUF, steady_body, 0)

        for i in range(tail_start, n_chunks):
            step(i, i % _NBUF, tail=(i + _LA >= n_chunks))
        for i in range(n_chunks - lag, n_chunks):
            wait_out(i, i % _NBUF)

    return gather_kernel(x, table)


def kernel(x, table):
    b, s = x.shape
    out = _sc_gather(x.astype(jnp.int32), table, b, s)
    return out.reshape(b, s, D_MODEL)
